# baseline (speedup 1.0000x reference)
import functools

import jax
import jax.numpy as jnp
from jax import lax
from jax.experimental import pallas as pl
from jax.experimental.pallas import tpu as pltpu


def _se_fused_kernel(x_ref, w1t_ref, w2t_ref, o_ref, *, hw, inv_hw):
    x = x_ref[...]
    col = lax.broadcasted_iota(jnp.int32, x.shape, 2)
    left = col < hw
    tot = jnp.sum(x, axis=2, dtype=jnp.float32)
    ev = jnp.sum(jnp.where(left, x, 0.0), axis=2, dtype=jnp.float32)
    pooled = jnp.concatenate([ev, tot - ev], axis=1) * inv_hw
    h = jnp.dot(pooled, w1t_ref[...], preferred_element_type=jnp.float32)
    h = jnp.maximum(h, 0.0)
    logits = jnp.dot(h, w2t_ref[...], preferred_element_type=jnp.float32)
    gate = pl.reciprocal(1.0 + jnp.exp(-logits), approx=True)
    chalf = x.shape[1]
    g_even = gate[:, :chalf, None]
    g_odd = gate[:, chalf:, None]
    o_ref[...] = x * jnp.where(left, g_even, g_odd)


@functools.partial(jax.jit, static_argnames=("bt",))
def _se_forward(x, w1t_p, w2t_p, bt):
    B, C, H, W = x.shape
    HW = H * W
    Cr = w1t_p.shape[1]
    x3 = x.reshape(B, C // 2, 2 * HW)
    out3 = pl.pallas_call(
        functools.partial(_se_fused_kernel, hw=HW, inv_hw=1.0 / HW),
        out_shape=jax.ShapeDtypeStruct((B, C // 2, 2 * HW), x.dtype),
        grid=(B // bt,),
        in_specs=[
            pl.BlockSpec((bt, C // 2, 2 * HW), lambda b: (b, 0, 0)),
            pl.BlockSpec((C, Cr), lambda b: (0, 0)),
            pl.BlockSpec((Cr, C), lambda b: (0, 0)),
        ],
        out_specs=pl.BlockSpec((bt, C // 2, 2 * HW), lambda b: (b, 0, 0)),
        compiler_params=pltpu.CompilerParams(
            dimension_semantics=("parallel",),
            vmem_limit_bytes=100 << 20,
        ),
    )(x3, w1t_p, w2t_p)
    return out3.reshape(B, C, H, W)


def kernel(x, w1, w2):
    C = x.shape[1]
    perm = jnp.concatenate([jnp.arange(0, C, 2), jnp.arange(1, C, 2)])
    w1t_p = w1.T[perm, :]
    w2t_p = w2.T[:, perm]
    return _se_forward(x, w1t_p, w2t_p, bt=2)
